# baseline (speedup 1.0000x reference)
import functools

import jax
import jax.numpy as jnp
from jax import lax
from jax.experimental import pallas as pl
from jax.experimental.pallas import tpu as pltpu
from jax.experimental.pallas import tpu_sc as plsc

N = 10000
C = 128
E = 320000
NC = 2
NS = 16
NW = NC * NS
EPW = E // NW
CH = 128
NCH = 80
PAD = NCH * CH - EPW
NPAD = 10112
RPS = NPAD // NS
NHALF = NCH // 2

_mesh = plsc.VectorSubcoreMesh(core_axis_name="c", subcore_axis_name="s")


@functools.partial(
    pl.kernel,
    out_type=jax.ShapeDtypeStruct((NC * NPAD,), jnp.float32),
    mesh=_mesh,
    scratch_types=[
        pltpu.VMEM((NCH, CH), jnp.int32),
        pltpu.VMEM((CH,), jnp.float32),
        pltpu.VMEM_SHARED((NPAD,), jnp.float32),
    ],
)
def _sc_degree(col_hbm, zeros_hbm, ones_hbm, deg_out, col_v, ones_v, deg_sh):
    c = lax.axis_index("c")
    s = lax.axis_index("s")
    wid = s * NC + c

    @pl.when(s == 0)
    def _():
        pltpu.sync_copy(zeros_hbm, deg_sh)

    pltpu.sync_copy(col_hbm.at[pl.ds(wid * NCH, NCH)], col_v)
    pltpu.sync_copy(ones_hbm, ones_v)
    plsc.subcore_barrier()

    def body(j, carry):
        pltpu.sync_copy(ones_v, deg_sh.at[col_v.at[j]], add=True)
        return carry

    lax.fori_loop(0, NCH, body, 0)
    plsc.subcore_barrier()

    @pl.when(s == 0)
    def _():
        pltpu.sync_copy(deg_sh, deg_out.at[pl.ds(c * NPAD, NPAD)])


@functools.partial(
    pl.kernel,
    out_type=jax.ShapeDtypeStruct((NC, NPAD, C), jnp.float32),
    mesh=_mesh,
    scratch_types=[
        pltpu.VMEM((NHALF, CH), jnp.int32),
        pltpu.VMEM((NHALF, CH), jnp.int32),
        pltpu.VMEM((CH, C), jnp.float32),
        pltpu.VMEM((CH, C), jnp.float32),
        pltpu.VMEM_SHARED((NPAD, C), jnp.float32),
        pltpu.SemaphoreType.DMA,
        pltpu.SemaphoreType.DMA,
        pltpu.SemaphoreType.DMA,
        pltpu.SemaphoreType.DMA,
    ],
)
def _sc_aggregate(hs_hbm, row_hbm, col_hbm, zeros_hbm, acc_out,
                  row_v, col_v, rows0, rows1, acc_sh, sem0, sem1, scb0, scb1):
    c = lax.axis_index("c")
    s = lax.axis_index("s")
    wid = s * NC + c

    HC = CH // 2
    dummy = hs_hbm.at[pl.ds(0, HC)]

    zdesc = pltpu.async_copy(zeros_hbm.at[pl.ds(s * RPS, RPS)],
                             acc_sh.at[pl.ds(s * RPS, RPS)], scb0)

    def _gather(j, dst, semA, semB):
        pltpu.async_copy(hs_hbm.at[row_v.at[j, pl.ds(0, HC)]],
                         dst.at[pl.ds(0, HC)], semA)
        pltpu.async_copy(hs_hbm.at[row_v.at[j, pl.ds(HC, HC)]],
                         dst.at[pl.ds(HC, HC)], semB)

    def _gwait(dst, semA, semB):
        pltpu.make_async_copy(dummy, dst.at[pl.ds(0, HC)], semA).wait()
        pltpu.make_async_copy(dummy, dst.at[pl.ds(HC, HC)], semB).wait()

    for h in range(NCH // NHALF):
        pltpu.sync_copy(row_hbm.at[pl.ds(wid * NCH + h * NHALF, NHALF)], row_v)
        pltpu.sync_copy(col_hbm.at[pl.ds(wid * NCH + h * NHALF, NHALF)], col_v)
        if h == 0:
            zdesc.wait()
            plsc.subcore_barrier()
        _gather(0, rows0, sem0, scb0)
        _gather(1, rows1, sem1, scb1)

        def body(i, carry):
            c0 = 2 * i
            c1 = c0 + 1
            _gwait(rows0, sem0, scb0)
            pltpu.sync_copy(rows0, acc_sh.at[col_v.at[c0]], add=True)

            @pl.when(c0 + 2 < NHALF)
            def _():
                _gather(c0 + 2, rows0, sem0, scb0)

            _gwait(rows1, sem1, scb1)
            pltpu.sync_copy(rows1, acc_sh.at[col_v.at[c1]], add=True)

            @pl.when(c1 + 2 < NHALF)
            def _():
                _gather(c1 + 2, rows1, sem1, scb1)

            return carry

        lax.fori_loop(0, NHALF // 2, body, 0)

    plsc.subcore_barrier()

    pltpu.sync_copy(acc_sh.at[pl.ds(s * RPS, RPS)],
                    acc_out.at[c, pl.ds(s * RPS, RPS)])


R = 1000
G = N // R


def _dinv_of(deg_blk):
    return lax.rsqrt(deg_blk[:, 0:1] + deg_blk[:, 1:2] + 1.0)


def _tc1_body(x_ref, deg_ref, w_ref, o_ref):
    dinv = _dinv_of(deg_ref[...])
    h = jnp.dot(x_ref[...], w_ref[...], preferred_element_type=jnp.float32)
    o_ref[...] = h * dinv


_tc_matmul_scale = pl.pallas_call(
    _tc1_body,
    grid=(G,),
    in_specs=[
        pl.BlockSpec((R, C), lambda i: (i, 0)),
        pl.BlockSpec((R, 2), lambda i: (i, 0)),
        pl.BlockSpec((C, C), lambda i: (0, 0)),
    ],
    out_specs=pl.BlockSpec((R, C), lambda i: (i, 0)),
    out_shape=jax.ShapeDtypeStruct((N, C), jnp.float32),
)


def _norm_block(a0, a1, hs, dinv, b, g, be):
    t = (a0 + a1 + hs) * dinv + b
    mu = jnp.mean(t, axis=-1, keepdims=True)
    d = t - mu
    var = jnp.mean(d * d, axis=-1, keepdims=True)
    y = d * lax.rsqrt(var + 1e-5) * g + be
    return jnp.maximum(y, 0.0)


def _tc2_body(a0_ref, a1_ref, hs_ref, deg_ref, b_ref, g_ref, be_ref, w_ref,
              o_ref):
    dinv = _dinv_of(deg_ref[...])
    y = _norm_block(a0_ref[0], a1_ref[0], hs_ref[...], dinv,
                    b_ref[...], g_ref[...], be_ref[...])
    o_ref[...] = jnp.dot(y, w_ref[...],
                         preferred_element_type=jnp.float32) * dinv


_tc_norm_matmul = pl.pallas_call(
    _tc2_body,
    grid=(G,),
    in_specs=[
        pl.BlockSpec((1, R, C), lambda i: (0, i, 0)),
        pl.BlockSpec((1, R, C), lambda i: (1, i, 0)),
        pl.BlockSpec((R, C), lambda i: (i, 0)),
        pl.BlockSpec((R, 2), lambda i: (i, 0)),
        pl.BlockSpec((1, C), lambda i: (0, 0)),
        pl.BlockSpec((1, C), lambda i: (0, 0)),
        pl.BlockSpec((1, C), lambda i: (0, 0)),
        pl.BlockSpec((C, C), lambda i: (0, 0)),
    ],
    out_specs=pl.BlockSpec((R, C), lambda i: (i, 0)),
    out_shape=jax.ShapeDtypeStruct((N, C), jnp.float32),
)


def _tc3_body(a0_ref, a1_ref, hs_ref, deg_ref, b_ref, g_ref, be_ref, o_ref):
    dinv = _dinv_of(deg_ref[...])
    o_ref[...] = _norm_block(a0_ref[0], a1_ref[0], hs_ref[...], dinv,
                             b_ref[...], g_ref[...], be_ref[...])


_tc_norm_out = pl.pallas_call(
    _tc3_body,
    grid=(G,),
    in_specs=[
        pl.BlockSpec((1, R, C), lambda i: (0, i, 0)),
        pl.BlockSpec((1, R, C), lambda i: (1, i, 0)),
        pl.BlockSpec((R, C), lambda i: (i, 0)),
        pl.BlockSpec((R, 2), lambda i: (i, 0)),
        pl.BlockSpec((1, C), lambda i: (0, 0)),
        pl.BlockSpec((1, C), lambda i: (0, 0)),
        pl.BlockSpec((1, C), lambda i: (0, 0)),
    ],
    out_specs=pl.BlockSpec((R, C), lambda i: (i, 0)),
    out_shape=jax.ShapeDtypeStruct((N, C), jnp.float32),
)


@jax.jit
def kernel(x, edge_index, W1, b1, gamma1, beta1, W2, b2, gamma2, beta2):
    row = edge_index[0].astype(jnp.int32)
    col = edge_index[1].astype(jnp.int32)
    k = jnp.arange(NW * PAD, dtype=jnp.int32)
    pad_r = (k % N).reshape(NW, PAD)
    pad_c = (N + (k % (NPAD - N))).reshape(NW, PAD)
    row2d = jnp.concatenate([row.reshape(NW, EPW), pad_r], axis=1)
    row2d = row2d.reshape(NW * NCH, CH)
    col2d = jnp.concatenate([col.reshape(NW, EPW), pad_c], axis=1)
    col2d = col2d.reshape(NW * NCH, CH)

    zeros_nd = jnp.zeros((NPAD, C), jnp.float32)
    zeros_n = jnp.zeros((NPAD,), jnp.float32)
    ones_ch = jnp.ones((CH,), jnp.float32)

    deg = _sc_degree(col2d, zeros_n, ones_ch)
    deg2 = jnp.swapaxes(deg.reshape(NC, NPAD)[:, :N], 0, 1)

    b1r, g1r, be1r = b1.reshape(1, C), gamma1.reshape(1, C), beta1.reshape(1, C)
    b2r, g2r, be2r = b2.reshape(1, C), gamma2.reshape(1, C), beta2.reshape(1, C)

    hs1 = _tc_matmul_scale(x, deg2, W1)
    acc1 = _sc_aggregate(hs1, row2d, col2d, zeros_nd)
    hs2 = _tc_norm_matmul(acc1, acc1, hs1, deg2, b1r, g1r, be1r, W2)
    acc2 = _sc_aggregate(hs2, row2d, col2d, zeros_nd)
    return _tc_norm_out(acc2, acc2, hs2, deg2, b2r, g2r, be2r)


# speedup vs baseline: 1.0217x; 1.0217x over previous
import functools

import jax
import jax.numpy as jnp
from jax import lax
from jax.experimental import pallas as pl
from jax.experimental.pallas import tpu as pltpu
from jax.experimental.pallas import tpu_sc as plsc

N = 10000
C = 128
E = 320000
NC = 2
NS = 16
NW = NC * NS
EPW = E // NW
CH = 128
NCH = 80
PAD = NCH * CH - EPW
NPAD = 10112
RPS = NPAD // NS
NHALF = NCH // 2

_mesh = plsc.VectorSubcoreMesh(core_axis_name="c", subcore_axis_name="s")


@functools.partial(
    pl.kernel,
    out_type=jax.ShapeDtypeStruct((NC * NPAD,), jnp.float32),
    mesh=_mesh,
    scratch_types=[
        pltpu.VMEM((NCH, CH), jnp.int32),
        pltpu.VMEM((CH,), jnp.float32),
        pltpu.VMEM_SHARED((NPAD,), jnp.float32),
    ],
)
def _sc_degree(col_hbm, zeros_hbm, ones_hbm, deg_out, col_v, ones_v, deg_sh):
    c = lax.axis_index("c")
    s = lax.axis_index("s")
    wid = s * NC + c

    @pl.when(s == 0)
    def _():
        pltpu.sync_copy(zeros_hbm, deg_sh)

    pltpu.sync_copy(col_hbm.at[pl.ds(wid * NCH, NCH)], col_v)
    pltpu.sync_copy(ones_hbm, ones_v)
    plsc.subcore_barrier()

    def body(j, carry):
        pltpu.sync_copy(ones_v, deg_sh.at[col_v.at[j]], add=True)
        return carry

    lax.fori_loop(0, NCH, body, 0)
    plsc.subcore_barrier()

    @pl.when(s == 0)
    def _():
        pltpu.sync_copy(deg_sh, deg_out.at[pl.ds(c * NPAD, NPAD)])


@functools.partial(
    pl.kernel,
    out_type=jax.ShapeDtypeStruct((NC, NPAD, C), jnp.float32),
    mesh=_mesh,
    scratch_types=[
        pltpu.VMEM((NHALF, CH), jnp.int32),
        pltpu.VMEM((NHALF, CH), jnp.int32),
        pltpu.VMEM((CH, C), jnp.float32),
        pltpu.VMEM((CH, C), jnp.float32),
        pltpu.VMEM_SHARED((NPAD, C), jnp.float32),
        pltpu.SemaphoreType.DMA,
        pltpu.SemaphoreType.DMA,
        pltpu.SemaphoreType.DMA,
        pltpu.SemaphoreType.DMA,
    ],
)
def _sc_aggregate(hs_hbm, row_hbm, col_hbm, zeros_hbm, acc_out,
                  row_v, col_v, rows0, rows1, acc_sh, sem0, sem1, scb0, scb1):
    c = lax.axis_index("c")
    s = lax.axis_index("s")
    wid = s * NC + c

    dummy = hs_hbm.at[pl.ds(0, CH)]

    zdesc = pltpu.async_copy(zeros_hbm.at[pl.ds(s * RPS, RPS)],
                             acc_sh.at[pl.ds(s * RPS, RPS)], scb0)

    for h in range(NCH // NHALF):
        pltpu.sync_copy(row_hbm.at[pl.ds(wid * NCH + h * NHALF, NHALF)], row_v)
        pltpu.sync_copy(col_hbm.at[pl.ds(wid * NCH + h * NHALF, NHALF)], col_v)
        if h == 0:
            zdesc.wait()
            plsc.subcore_barrier()
        pltpu.async_copy(hs_hbm.at[row_v.at[0]], rows0, sem0)
        pltpu.async_copy(hs_hbm.at[row_v.at[1]], rows1, sem1)

        def body(i, carry):
            c0 = 2 * i
            c1 = c0 + 1
            pltpu.make_async_copy(dummy, rows0, sem0).wait()
            pltpu.sync_copy(rows0, acc_sh.at[col_v.at[c0]], add=True)

            @pl.when(c0 + 2 < NHALF)
            def _():
                pltpu.async_copy(hs_hbm.at[row_v.at[c0 + 2]], rows0, sem0)

            pltpu.make_async_copy(dummy, rows1, sem1).wait()
            pltpu.sync_copy(rows1, acc_sh.at[col_v.at[c1]], add=True)

            @pl.when(c1 + 2 < NHALF)
            def _():
                pltpu.async_copy(hs_hbm.at[row_v.at[c1 + 2]], rows1, sem1)

            return carry

        lax.fori_loop(0, NHALF // 2, body, 0)

    plsc.subcore_barrier()

    pltpu.sync_copy(acc_sh.at[pl.ds(s * RPS, RPS)],
                    acc_out.at[c, pl.ds(s * RPS, RPS)])


R = 1000
G = N // R


def _dinv_of(deg_blk):
    return lax.rsqrt(deg_blk[:, 0:1] + deg_blk[:, 1:2] + 1.0)


def _tc1_body(x_ref, deg_ref, w_ref, o_ref):
    dinv = _dinv_of(deg_ref[...])
    h = jnp.dot(x_ref[...], w_ref[...], preferred_element_type=jnp.float32)
    o_ref[...] = h * dinv


_tc_matmul_scale = pl.pallas_call(
    _tc1_body,
    grid=(G,),
    in_specs=[
        pl.BlockSpec((R, C), lambda i: (i, 0)),
        pl.BlockSpec((R, 2), lambda i: (i, 0)),
        pl.BlockSpec((C, C), lambda i: (0, 0)),
    ],
    out_specs=pl.BlockSpec((R, C), lambda i: (i, 0)),
    out_shape=jax.ShapeDtypeStruct((N, C), jnp.float32),
)


def _norm_block(a0, a1, hs, dinv, b, g, be):
    t = (a0 + a1 + hs) * dinv + b
    mu = jnp.mean(t, axis=-1, keepdims=True)
    d = t - mu
    var = jnp.mean(d * d, axis=-1, keepdims=True)
    y = d * lax.rsqrt(var + 1e-5) * g + be
    return jnp.maximum(y, 0.0)


def _tc2_body(a0_ref, a1_ref, hs_ref, deg_ref, b_ref, g_ref, be_ref, w_ref,
              o_ref):
    dinv = _dinv_of(deg_ref[...])
    y = _norm_block(a0_ref[0], a1_ref[0], hs_ref[...], dinv,
                    b_ref[...], g_ref[...], be_ref[...])
    o_ref[...] = jnp.dot(y, w_ref[...],
                         preferred_element_type=jnp.float32) * dinv


_tc_norm_matmul = pl.pallas_call(
    _tc2_body,
    grid=(G,),
    in_specs=[
        pl.BlockSpec((1, R, C), lambda i: (0, i, 0)),
        pl.BlockSpec((1, R, C), lambda i: (1, i, 0)),
        pl.BlockSpec((R, C), lambda i: (i, 0)),
        pl.BlockSpec((R, 2), lambda i: (i, 0)),
        pl.BlockSpec((1, C), lambda i: (0, 0)),
        pl.BlockSpec((1, C), lambda i: (0, 0)),
        pl.BlockSpec((1, C), lambda i: (0, 0)),
        pl.BlockSpec((C, C), lambda i: (0, 0)),
    ],
    out_specs=pl.BlockSpec((R, C), lambda i: (i, 0)),
    out_shape=jax.ShapeDtypeStruct((N, C), jnp.float32),
)


def _tc3_body(a0_ref, a1_ref, hs_ref, deg_ref, b_ref, g_ref, be_ref, o_ref):
    dinv = _dinv_of(deg_ref[...])
    o_ref[...] = _norm_block(a0_ref[0], a1_ref[0], hs_ref[...], dinv,
                             b_ref[...], g_ref[...], be_ref[...])


_tc_norm_out = pl.pallas_call(
    _tc3_body,
    grid=(G,),
    in_specs=[
        pl.BlockSpec((1, R, C), lambda i: (0, i, 0)),
        pl.BlockSpec((1, R, C), lambda i: (1, i, 0)),
        pl.BlockSpec((R, C), lambda i: (i, 0)),
        pl.BlockSpec((R, 2), lambda i: (i, 0)),
        pl.BlockSpec((1, C), lambda i: (0, 0)),
        pl.BlockSpec((1, C), lambda i: (0, 0)),
        pl.BlockSpec((1, C), lambda i: (0, 0)),
    ],
    out_specs=pl.BlockSpec((R, C), lambda i: (i, 0)),
    out_shape=jax.ShapeDtypeStruct((N, C), jnp.float32),
)


@jax.jit
def kernel(x, edge_index, W1, b1, gamma1, beta1, W2, b2, gamma2, beta2):
    row = edge_index[0].astype(jnp.int32)
    col = edge_index[1].astype(jnp.int32)
    k = jnp.arange(NW * PAD, dtype=jnp.int32)
    pad_r = (k % N).reshape(NW, PAD)
    pad_c = (N + (k % (NPAD - N))).reshape(NW, PAD)
    row2d = jnp.concatenate([row.reshape(NW, EPW), pad_r], axis=1)
    row2d = row2d.reshape(NW * NCH, CH)
    col2d = jnp.concatenate([col.reshape(NW, EPW), pad_c], axis=1)
    col2d = col2d.reshape(NW * NCH, CH)

    zeros_nd = jnp.zeros((NPAD, C), jnp.float32)
    zeros_n = jnp.zeros((NPAD,), jnp.float32)
    ones_ch = jnp.ones((CH,), jnp.float32)

    deg = _sc_degree(col2d, zeros_n, ones_ch)
    deg2 = jnp.swapaxes(deg.reshape(NC, NPAD)[:, :N], 0, 1)

    b1r, g1r, be1r = b1.reshape(1, C), gamma1.reshape(1, C), beta1.reshape(1, C)
    b2r, g2r, be2r = b2.reshape(1, C), gamma2.reshape(1, C), beta2.reshape(1, C)

    hs1 = _tc_matmul_scale(x, deg2, W1)
    acc1 = _sc_aggregate(hs1, row2d, col2d, zeros_nd)
    hs2 = _tc_norm_matmul(acc1, acc1, hs1, deg2, b1r, g1r, be1r, W2)
    acc2 = _sc_aggregate(hs2, row2d, col2d, zeros_nd)
    return _tc_norm_out(acc2, acc2, hs2, deg2, b2r, g2r, be2r)
